# Initial kernel scaffold; baseline (speedup 1.0000x reference)
import jax
import jax.numpy as jnp
from jax.experimental import pallas as pl


def kernel(pred, batch, sum_target, mean, std):
    raise NotImplementedError("write your pallas kernel here")


# speedup vs baseline: 3.2890x; 3.2890x over previous
import functools

import jax
import jax.numpy as jnp
from jax import lax
from jax.experimental import pallas as pl
from jax.experimental.pallas import tpu as pltpu
from jax.experimental.pallas import tpu_sc as plsc

_N = 6_400_000
_T = 4
_G = 512
_NC = 2
_NS = 16
_NW = _NC * _NS
_RPW = _N // _NW
_B = 8_000
_NB = _RPW // _B
_E = _B * _T
_GT = _G * 16
_GPW = _G // _NW
_BS_ITERS = 13

_mesh = plsc.VectorSubcoreMesh(core_axis_name="c", subcore_axis_name="s")
_params = pltpu.CompilerParams(needs_layout_passes=False)


def _wid():
    return lax.axis_index("s") * _NC + lax.axis_index("c")


def _sload(ref, i):
    return ref[pl.ds(i, 16)][0]


def _lower_bound(bbuf, x, lo0):
    def body(_, c):
        lo, hi = c
        mid = (lo + hi) >> 1
        act = lo < hi
        lt = _sload(bbuf, mid) < x
        lo = jnp.where(act & lt, mid + 1, lo)
        hi = jnp.where(act & (~lt), mid, hi)
        return lo, hi

    lo, _ = lax.fori_loop(0, _BS_ITERS, body, (lo0, jnp.int32(_B)))
    return lo


@functools.partial(
    pl.kernel,
    out_type=jax.ShapeDtypeStruct((_NW, _GT), jnp.float32),
    mesh=_mesh,
    compiler_params=_params,
    scratch_types=[
        pltpu.VMEM((_E,), jnp.float32),
        pltpu.VMEM((_B + 16,), jnp.int32),
        pltpu.VMEM((_GT,), jnp.float32),
    ],
)
def _partial_sums(pred_hbm, batch_hbm, out_hbm, pbuf, bbuf, table):
    wid = _wid()
    iota = lax.iota(jnp.int32, 16)
    zeros16 = jnp.zeros((16,), jnp.float32)

    def zero_body(g, carry):
        table[pl.ds(g * 16, 16)] = zeros16
        return carry

    lax.fori_loop(0, _G, zero_body, 0)

    def block_body(blk, carry):
        rbase = pl.multiple_of(wid * _RPW + blk * _B, _B)
        ebase = pl.multiple_of(rbase * _T, _E)
        pltpu.sync_copy(pred_hbm.at[pl.ds(ebase, _E)], pbuf)
        pltpu.sync_copy(batch_hbm.at[pl.ds(rbase, _B)], bbuf.at[pl.ds(0, _B)])

        g0 = _sload(bbuf, 0)
        g1 = _sload(bbuf, _B - 1)

        def run_body(r, p):
            g = g0 + r
            q = _lower_bound(bbuf, g + 1, p)
            e0 = p * 4
            e1 = q * 4
            ha = e0 & ~15
            nv = (e1 + 15 - ha) >> 4

            def vloop(k, acc):
                i = ha + k * 16
                lane = i + iota
                m = (lane >= e0) & (lane < e1)
                return acc + jnp.where(m, pbuf[pl.ds(i, 16)], 0.0)

            acc = lax.fori_loop(0, nv, vloop, zeros16)
            s0 = jnp.sum(jnp.where(iota % 4 == 0, acc, 0.0))
            s1 = jnp.sum(jnp.where(iota % 4 == 1, acc, 0.0))
            s2 = jnp.sum(jnp.where(iota % 4 == 2, acc, 0.0))
            s3 = jnp.sum(jnp.where(iota % 4 == 3, acc, 0.0))
            cnt = (q - p).astype(jnp.float32)
            upd = jnp.where(
                iota == 0, s0,
                jnp.where(iota == 1, s1,
                          jnp.where(iota == 2, s2,
                                    jnp.where(iota == 3, s3,
                                              jnp.where(iota == 4, cnt,
                                                        0.0)))))
            table[pl.ds(g * 16, 16)] = table[pl.ds(g * 16, 16)] + upd
            return q

        lax.fori_loop(0, g1 - g0 + 1, run_body, jnp.int32(0))
        return carry

    lax.fori_loop(0, _NB, block_body, 0)
    pltpu.sync_copy(table, out_hbm.at[wid])


@functools.partial(
    pl.kernel,
    out_type=jax.ShapeDtypeStruct((_GT,), jnp.float32),
    mesh=_mesh,
    compiler_params=_params,
    scratch_types=[
        pltpu.VMEM((_NW * _GPW * 16,), jnp.float32),
        pltpu.VMEM((_GPW * 16,), jnp.float32),
        pltpu.VMEM((32,), jnp.float32),
        pltpu.VMEM((_GPW * 16,), jnp.float32),
        pltpu.SemaphoreType.DMA,
    ],
)
def _correction(part_hbm, st_hbm, ms_hbm, corr_hbm, part_v, st_v, ms_v,
                out_v, sem):
    wid = _wid()
    iota = lax.iota(jnp.int32, 16)
    i4 = iota % 4
    seg = _GPW * 16

    handles = []
    for w2 in range(_NW):
        handles.append(pltpu.async_copy(
            part_hbm.at[pl.ds(w2 * _GT + wid * seg, seg)],
            part_v.at[pl.ds(w2 * seg, seg)], sem))
    handles.append(pltpu.async_copy(st_hbm.at[pl.ds(wid * seg, seg)], st_v,
                                    sem))
    handles.append(pltpu.async_copy(ms_hbm, ms_v, sem))
    for h in handles:
        h.wait()

    meanv_ = None

    def seg_body(j, carry):
        def add_w(w2, acc):
            return acc + part_v[pl.ds(w2 * seg + j * 16, 16)]

        acc = lax.fori_loop(0, _NW, add_w, jnp.zeros((16,), jnp.float32))
        s0 = jnp.sum(jnp.where(iota == 0, acc, 0.0))
        s1 = jnp.sum(jnp.where(iota == 1, acc, 0.0))
        s2 = jnp.sum(jnp.where(iota == 2, acc, 0.0))
        s3 = jnp.sum(jnp.where(iota == 3, acc, 0.0))
        cnt = jnp.sum(jnp.where(iota == 4, acc, 0.0))
        psum = jnp.where(i4 == 0, s0,
                         jnp.where(i4 == 1, s1,
                                   jnp.where(i4 == 2, s2, s3)))
        st = st_v[pl.ds(j * 16, 16)]
        meanv = ms_v[pl.ds(0, 16)]
        stdv = ms_v[pl.ds(16, 16)]
        corr = ((st - cnt * meanv) / stdv - psum) / cnt
        out_v[pl.ds(j * 16, 16)] = corr
        return carry

    lax.fori_loop(0, _GPW, seg_body, 0)
    pltpu.sync_copy(out_v, corr_hbm.at[pl.ds(wid * seg, seg)])


@functools.partial(
    pl.kernel,
    out_type=jax.ShapeDtypeStruct((_N * _T,), jnp.float32),
    mesh=_mesh,
    compiler_params=_params,
    scratch_types=[
        pltpu.VMEM((_E,), jnp.float32),
        pltpu.VMEM((_B + 16,), jnp.int32),
        pltpu.VMEM((_GT,), jnp.float32),
    ],
)
def _apply(pred_hbm, batch_hbm, corr_hbm, out_hbm, pbuf, bbuf, corr_v):
    wid = _wid()
    iota = lax.iota(jnp.int32, 16)
    pltpu.sync_copy(corr_hbm, corr_v)

    def block_body(blk, carry):
        rbase = pl.multiple_of(wid * _RPW + blk * _B, _B)
        ebase = pl.multiple_of(rbase * _T, _E)
        pltpu.sync_copy(pred_hbm.at[pl.ds(ebase, _E)], pbuf)
        pltpu.sync_copy(batch_hbm.at[pl.ds(rbase, _B)], bbuf.at[pl.ds(0, _B)])

        g0 = _sload(bbuf, 0)
        g1 = _sload(bbuf, _B - 1)

        def run_body(r, p):
            g = g0 + r
            q = _lower_bound(bbuf, g + 1, p)
            e0 = p * 4
            e1 = q * 4
            ha = e0 & ~15
            nv = (e1 + 15 - ha) >> 4
            cvec = corr_v[pl.ds(g * 16, 16)]

            def vloop(k, carry2):
                i = ha + k * 16
                lane = i + iota
                m = (lane >= e0) & (lane < e1)
                pbuf[pl.ds(i, 16)] = (
                    pbuf[pl.ds(i, 16)] + jnp.where(m, cvec, 0.0))
                return carry2

            lax.fori_loop(0, nv, vloop, 0)
            return q

        lax.fori_loop(0, g1 - g0 + 1, run_body, jnp.int32(0))
        pltpu.sync_copy(pbuf, out_hbm.at[pl.ds(ebase, _E)])
        return carry

    lax.fori_loop(0, _NB, block_body, 0)


def kernel(pred, batch, sum_target, mean, std):
    pred_flat = pred.reshape(-1)
    st16 = jnp.tile(sum_target, (1, 4)).reshape(-1)
    ms = jnp.concatenate([jnp.tile(mean, 4), jnp.tile(std, 4)])
    part = _partial_sums(pred_flat, batch)
    corr = _correction(part.reshape(-1), st16, ms)
    out = _apply(pred_flat, batch, corr)
    return out.reshape(_N, _T)


# speedup vs baseline: 143.3001x; 43.5694x over previous
import functools

import jax
import jax.numpy as jnp
from jax import lax
from jax.experimental import pallas as pl
from jax.experimental.pallas import tpu as pltpu
from jax.experimental.pallas import tpu_sc as plsc

_N = 6_400_000
_T = 4
_G = 512
_NC = 2
_NS = 16
_NW = _NC * _NS
_TILE = 128
_NTILES = _N // _TILE
_TPB = 50
_BR = _TPB * _TILE
_BE = _BR * _T
_NBLK = _NTILES // _TPB
_BLK_Q, _BLK_R = divmod(_NBLK, _NW)
_GT = _G * 16
_GPW = _G // _NW
_BS_ITERS = 13

_mesh = plsc.VectorSubcoreMesh(core_axis_name="c", subcore_axis_name="s")
_params = pltpu.CompilerParams(needs_layout_passes=False)


def _wid():
    return lax.axis_index("s") * _NC + lax.axis_index("c")


def _sload(ref, i):
    return ref[pl.ds(i, 16)][0]


def _lower_bound(bbuf, x, lo0):
    def body(_, c):
        lo, hi = c
        mid = (lo + hi) >> 1
        act = lo < hi
        lt = _sload(bbuf, mid) < x
        lo = jnp.where(act & lt, mid + 1, lo)
        hi = jnp.where(act & (~lt), mid, hi)
        return lo, hi

    lo, _ = lax.fori_loop(0, _BS_ITERS, body, (lo0, jnp.int32(_BR)))
    return lo


def _run_bounds(p, q):
    t0 = p >> 7
    t1 = jnp.maximum(q - 1, p) >> 7
    hi1 = jnp.minimum(q, (t0 + 1) * _TILE)
    lo2 = jnp.where(t1 > t0, t1 * _TILE, q)
    return t0, t1, hi1, lo2


@functools.partial(
    pl.kernel,
    out_type=jax.ShapeDtypeStruct((_NW * _GT,), jnp.float32),
    mesh=_mesh,
    compiler_params=_params,
    scratch_types=[
        pltpu.VMEM((_BE,), jnp.float32),
        pltpu.VMEM((_BR + 16,), jnp.int32),
        pltpu.VMEM((_GT,), jnp.float32),
    ],
)
def _partial_sums(pred_hbm, batch_hbm, out_hbm, pbuf, bbuf, table):
    wid = _wid()
    iota = lax.iota(jnp.int32, 16)
    zeros16 = jnp.zeros((16,), jnp.float32)

    def zero_body(g, carry):
        table[pl.ds(g * 16, 16)] = zeros16
        return carry

    lax.fori_loop(0, _G, zero_body, 0)

    def masked_tile(tt, lo, hi, accs):
        base = tt * 512
        out = list(accs)
        for v in range(8):
            rows = tt * _TILE + v * 16 + iota
            m = (rows >= lo) & (rows < hi)
            for j in range(_T):
                out[j] = out[j] + jnp.where(
                    m, pbuf[pl.ds(base + j * _TILE + v * 16, 16)], 0.0)
        return tuple(out)

    def full_tile(tt, accs):
        base = tt * 512
        out = list(accs)
        for j in range(_T):
            for v in range(8):
                out[j] = out[j] + pbuf[pl.ds(base + j * _TILE + v * 16, 16)]
        return tuple(out)

    def block_body(k, carry):
        blk = wid + k * _NW
        rbase = pl.multiple_of(blk * _BR, _BR)
        ebase = pl.multiple_of(rbase * _T, _BE)
        pltpu.sync_copy(pred_hbm.at[pl.ds(ebase, _BE)], pbuf)
        pltpu.sync_copy(batch_hbm.at[pl.ds(rbase, _BR)],
                        bbuf.at[pl.ds(0, _BR)])

        g0 = _sload(bbuf, 0)
        g1 = _sload(bbuf, _BR - 1)

        def run_body(r, p):
            g = g0 + r
            q = _lower_bound(bbuf, g + 1, p)
            t0, t1, hi1, lo2 = _run_bounds(p, q)
            accs = (zeros16, zeros16, zeros16, zeros16)
            accs = masked_tile(t0, p, hi1, accs)
            accs = lax.fori_loop(t0 + 1, t1, full_tile, accs)
            accs = masked_tile(t1, lo2, q, accs)
            s0, s1, s2, s3 = (jnp.sum(a) for a in accs)
            cnt = (q - p).astype(jnp.float32)
            upd = jnp.where(
                iota == 0, s0,
                jnp.where(iota == 1, s1,
                          jnp.where(iota == 2, s2,
                                    jnp.where(iota == 3, s3,
                                              jnp.where(iota == 4, cnt,
                                                        0.0)))))
            table[pl.ds(g * 16, 16)] = table[pl.ds(g * 16, 16)] + upd
            return q

        lax.fori_loop(0, g1 - g0 + 1, run_body, jnp.int32(0))
        return carry

    nblk = _BLK_Q + (wid < _BLK_R).astype(jnp.int32)
    lax.fori_loop(0, nblk, block_body, 0)
    pltpu.sync_copy(table, out_hbm.at[pl.ds(wid * _GT, _GT)])


@functools.partial(
    pl.kernel,
    out_type=jax.ShapeDtypeStruct((_GT,), jnp.float32),
    mesh=_mesh,
    compiler_params=_params,
    scratch_types=[
        pltpu.VMEM((_NW * _GPW * 16,), jnp.float32),
        pltpu.VMEM((_GPW * 16,), jnp.float32),
        pltpu.VMEM((32,), jnp.float32),
        pltpu.VMEM((_GPW * 16,), jnp.float32),
        pltpu.SemaphoreType.DMA,
    ],
)
def _correction(part_hbm, st_hbm, ms_hbm, corr_hbm, part_v, st_v, ms_v,
                out_v, sem):
    wid = _wid()
    iota = lax.iota(jnp.int32, 16)
    i4 = iota % 4
    seg = _GPW * 16

    handles = []
    for w2 in range(_NW):
        handles.append(pltpu.async_copy(
            part_hbm.at[pl.ds(w2 * _GT + wid * seg, seg)],
            part_v.at[pl.ds(w2 * seg, seg)], sem))
    handles.append(pltpu.async_copy(st_hbm.at[pl.ds(wid * seg, seg)], st_v,
                                    sem))
    handles.append(pltpu.async_copy(ms_hbm, ms_v, sem))
    for h in handles:
        h.wait()

    def seg_body(j, carry):
        def add_w(w2, acc):
            return acc + part_v[pl.ds(w2 * seg + j * 16, 16)]

        acc = lax.fori_loop(0, _NW, add_w, jnp.zeros((16,), jnp.float32))
        s0 = jnp.sum(jnp.where(iota == 0, acc, 0.0))
        s1 = jnp.sum(jnp.where(iota == 1, acc, 0.0))
        s2 = jnp.sum(jnp.where(iota == 2, acc, 0.0))
        s3 = jnp.sum(jnp.where(iota == 3, acc, 0.0))
        cnt = jnp.sum(jnp.where(iota == 4, acc, 0.0))
        psum = jnp.where(i4 == 0, s0,
                         jnp.where(i4 == 1, s1,
                                   jnp.where(i4 == 2, s2, s3)))
        st = st_v[pl.ds(j * 16, 16)]
        meanv = ms_v[pl.ds(0, 16)]
        stdv = ms_v[pl.ds(16, 16)]
        corr = ((st - cnt * meanv) / stdv - psum) / cnt
        out_v[pl.ds(j * 16, 16)] = corr
        return carry

    lax.fori_loop(0, _GPW, seg_body, 0)
    pltpu.sync_copy(out_v, corr_hbm.at[pl.ds(wid * seg, seg)])


@functools.partial(
    pl.kernel,
    out_type=jax.ShapeDtypeStruct((_N * _T,), jnp.float32),
    mesh=_mesh,
    compiler_params=_params,
    scratch_types=[
        pltpu.VMEM((_BE,), jnp.float32),
        pltpu.VMEM((_BR + 16,), jnp.int32),
        pltpu.VMEM((_GT,), jnp.float32),
    ],
)
def _apply(pred_hbm, batch_hbm, corr_hbm, out_hbm, pbuf, bbuf, corr_v):
    wid = _wid()
    iota = lax.iota(jnp.int32, 16)
    pltpu.sync_copy(corr_hbm, corr_v)

    def masked_tile(tt, lo, hi, cj):
        base = tt * 512
        for v in range(8):
            rows = tt * _TILE + v * 16 + iota
            m = (rows >= lo) & (rows < hi)
            for j in range(_T):
                off = base + j * _TILE + v * 16
                pbuf[pl.ds(off, 16)] = (
                    pbuf[pl.ds(off, 16)] + jnp.where(m, cj[j], 0.0))

    def block_body(k, carry):
        blk = wid + k * _NW
        rbase = pl.multiple_of(blk * _BR, _BR)
        ebase = pl.multiple_of(rbase * _T, _BE)
        pltpu.sync_copy(pred_hbm.at[pl.ds(ebase, _BE)], pbuf)
        pltpu.sync_copy(batch_hbm.at[pl.ds(rbase, _BR)],
                        bbuf.at[pl.ds(0, _BR)])

        g0 = _sload(bbuf, 0)
        g1 = _sload(bbuf, _BR - 1)

        def run_body(r, p):
            g = g0 + r
            q = _lower_bound(bbuf, g + 1, p)
            t0, t1, hi1, lo2 = _run_bounds(p, q)
            cvec = corr_v[pl.ds(g * 16, 16)]
            cj = tuple(jnp.full((16,), cvec[j], jnp.float32)
                       for j in range(_T))

            masked_tile(t0, p, hi1, cj)

            def full_tile(tt, carry2):
                base = tt * 512
                for j in range(_T):
                    for v in range(8):
                        off = base + j * _TILE + v * 16
                        pbuf[pl.ds(off, 16)] = pbuf[pl.ds(off, 16)] + cj[j]
                return carry2

            lax.fori_loop(t0 + 1, t1, full_tile, 0)
            masked_tile(t1, lo2, q, cj)
            return q

        lax.fori_loop(0, g1 - g0 + 1, run_body, jnp.int32(0))
        pltpu.sync_copy(pbuf, out_hbm.at[pl.ds(ebase, _BE)])
        return carry

    nblk = _BLK_Q + (wid < _BLK_R).astype(jnp.int32)
    lax.fori_loop(0, nblk, block_body, 0)


def kernel(pred, batch, sum_target, mean, std):
    pred_flat = pred.reshape(_NTILES, _TILE, _T).transpose(0, 2, 1).reshape(-1)
    st16 = jnp.tile(sum_target, (1, 4)).reshape(-1)
    ms = jnp.concatenate([jnp.tile(mean, 4), jnp.tile(std, 4)])
    part = _partial_sums(pred_flat, batch)
    corr = _correction(part, st16, ms)
    out = _apply(pred_flat, batch, corr)
    return out.reshape(_NTILES, _T, _TILE).transpose(0, 2, 1).reshape(_N, _T)
